# baseline (speedup 1.0000x reference)
import functools

import jax
import jax.numpy as jnp
from jax import lax
from jax.experimental import pallas as pl
from jax.experimental.pallas import tpu as pltpu
from jax.experimental.pallas import tpu_sc as plsc

_K = 64
_SIGN = -2147483648
_CAP = 1024
_NSAMP = 512
_RTARG = 6
_LANES = 16

_info = plsc.get_sparse_core_info()
_NW = _info.num_cores * _info.num_subcores


def _key16(ref, off):
    iv = lax.bitcast_convert_type(ref[pl.ds(off, _LANES)], jnp.int32)
    return iv ^ (lax.shift_right_arithmetic(iv, 31) & jnp.int32(0x7FFFFFFF))


def _lane_total(v, rot_idx):
    for idx in rot_idx:
        v = v + v.at[idx].get(mode="promise_in_bounds")
    return v[0]


def _row_threshold(rowbuf, candbuf, n, rot_idx):
    nv = n // _LANES
    one, zero = jnp.int32(1), jnp.int32(0)
    zvec = jnp.zeros((_LANES,), jnp.int32)

    def counts3(load_key, trips, unroll, c3s, c2s, c1s):
        def body(i, st):
            a3, a2, a1 = st
            base = i * (_LANES * unroll)
            for u in range(unroll):
                k = load_key(base + u * _LANES)
                a3 = a3 + jnp.where(k >= c3s, one, zero)
                a2 = a2 + jnp.where(k >= c2s, one, zero)
                a1 = a1 + jnp.where(k >= c1s, one, zero)
            return (a3, a2, a1)
        a3, a2, a1 = lax.fori_loop(0, trips, body, (zvec, zvec, zvec))
        return (_lane_total(a3, rot_idx), _lane_total(a2, rot_idx),
                _lane_total(a1, rot_idx))

    def step2bit(t, st, load_key, ntrips, unroll, stop, kmin):
        upfx, cnt = st
        trips = jnp.where(cnt > stop, ntrips, 0)
        b1 = lax.shift_left(jnp.int32(1), jnp.int32(31) - 2 * t)
        b0 = lax.shift_left(jnp.int32(1), jnp.int32(30) - 2 * t)
        c3 = upfx | b1 | b0
        c2 = upfx | b1
        c1 = upfx | b0
        n3, n2, n1 = counts3(load_key, trips, unroll,
                             c3 ^ _SIGN, c2 ^ _SIGN, c1 ^ _SIGN)
        take3 = n3 >= kmin
        take2 = jnp.logical_and(~take3, n2 >= kmin)
        take1 = jnp.logical_and(~(take3 | take2), n1 >= kmin)
        newp = jnp.where(take3, c3,
                         jnp.where(take2, c2, jnp.where(take1, c1, upfx)))
        newc = jnp.where(take3, n3,
                         jnp.where(take2, n2, jnp.where(take1, n1, cnt)))
        return (newp, newc)

    stride = (nv // (_NSAMP // _LANES)) * _LANES

    def samp_key(off):
        return _key16(rowbuf, (off // _LANES) * stride)

    piv, _ = lax.fori_loop(
        0, 16,
        lambda t, st: step2bit(t, st, samp_key, _NSAMP // _LANES, 1,
                               _RTARG, _RTARG),
        (jnp.int32(0), jnp.int32(_NSAMP)))

    limit = jnp.int32(_CAP * _LANES)
    piv_s = piv ^ _SIGN

    def gbody(g, st):
        off, tot = st
        ks, hs = [], []
        for u in range(4):
            k = _key16(rowbuf, (g * 4 + u) * _LANES)
            m = k >= piv_s
            ks.append(jnp.where(m, k, jnp.int32(_SIGN)))
            hs.append(_lane_total(jnp.where(m, one, zero), rot_idx))
        for u in range(4):
            candbuf[pl.ds(off, _LANES)] = ks[u]
            off = jnp.where(hs[u] > 0,
                            jnp.minimum(off + _LANES, limit), off)
            tot = tot + hs[u]
        return (off, tot)

    off_p, cnt_p = lax.fori_loop(0, nv // 4, gbody,
                                 (jnp.int32(0), jnp.int32(0)))
    ok = jnp.logical_and(cnt_p >= _K, cnt_p <= _CAP)

    def fb_key(off):
        return _key16(rowbuf, off)

    fb_trips = jnp.where(ok, 0, nv // 4)
    upfx_fb, cnt_fb = lax.fori_loop(
        0, 16,
        lambda t, st: step2bit(t, st, fb_key, fb_trips, 4, _CAP, _K),
        (jnp.int32(0), jnp.int32(n)))
    valid_fb = jnp.logical_and(~ok, cnt_fb <= _CAP)
    thr_fb = upfx_fb ^ _SIGN

    def cbody(i, off):
        k = _key16(rowbuf, i * _LANES)
        m = k >= thr_fb
        candbuf[pl.ds(off, _LANES)] = jnp.where(m, k, jnp.int32(_SIGN))
        hits = _lane_total(jnp.where(m, one, zero), rot_idx)
        return jnp.where(hits > 0, off + _LANES, off)

    off_fb = lax.fori_loop(0, jnp.where(valid_fb, nv, 0), cbody,
                           jnp.int32(0))

    use_fb = ~ok
    upfx_in = jnp.where(use_fb, upfx_fb, jnp.int32(0))
    cnt_in = jnp.where(use_fb, cnt_fb, cnt_p)
    sv = jnp.where(use_fb, off_fb, off_p) // _LANES
    sv = jnp.where(jnp.logical_and(use_fb, ~valid_fb), 0, sv)

    def cand_key(off):
        return candbuf[pl.ds(off, _LANES)]

    upfx, fcnt = lax.fori_loop(
        0, 16,
        lambda t, st: step2bit(t, st, cand_key, sv, 1, _K, _K),
        (upfx_in, cnt_in))

    pivhit = jnp.logical_and(ok, cnt_p == _K)
    upfx = jnp.where(pivhit, piv, upfx)
    fcnt = jnp.where(pivhit, cnt_p, fcnt)
    thr_s = upfx ^ _SIGN

    tie = fcnt > _K
    tie_trips = jnp.where(tie, nv // 4, 0)

    def gtbody(i, a):
        base = i * (_LANES * 4)
        for u in range(4):
            k = _key16(rowbuf, base + u * _LANES)
            a = a + jnp.where(k > thr_s, one, zero)
        return a

    ngt = _lane_total(lax.fori_loop(0, tie_trips, gtbody, zvec), rot_idx)
    req = _K - ngt

    lane = lax.iota(jnp.int32, _LANES)

    def idx_step(t, vpfx):
        b1 = lax.shift_left(jnp.int32(1), jnp.int32(15) - 2 * t)
        b0 = lax.shift_left(jnp.int32(1), jnp.int32(14) - 2 * t)
        c3, c2, c1 = vpfx | b1 | b0, vpfx | b1, vpfx | b0

        def body(i, st):
            a3, a2, a1 = st
            base = i * (_LANES * 4)
            for u in range(4):
                off = base + u * _LANES
                k = _key16(rowbuf, off)
                eq = k == thr_s
                rv = jnp.int32(n - 1) - (off + lane)
                a3 = a3 + jnp.where(jnp.logical_and(eq, rv >= c3), one, zero)
                a2 = a2 + jnp.where(jnp.logical_and(eq, rv >= c2), one, zero)
                a1 = a1 + jnp.where(jnp.logical_and(eq, rv >= c1), one, zero)
            return (a3, a2, a1)

        a3, a2, a1 = lax.fori_loop(0, tie_trips, body, (zvec, zvec, zvec))
        n3 = _lane_total(a3, rot_idx)
        n2 = _lane_total(a2, rot_idx)
        n1 = _lane_total(a1, rot_idx)
        take3 = n3 >= req
        take2 = jnp.logical_and(~take3, n2 >= req)
        take1 = jnp.logical_and(~(take3 | take2), n1 >= req)
        return jnp.where(take3, c3,
                         jnp.where(take2, c2, jnp.where(take1, c1, vpfx)))

    vstar = lax.fori_loop(0, 8, idx_step, jnp.int32(0))
    istar = jnp.where(tie, jnp.int32(n - 1) - vstar, jnp.int32(n - 1))

    return thr_s, istar


def _sc_thresholds(x):
    B, N = x.shape
    rows_per_w = B // _NW
    mesh = plsc.VectorSubcoreMesh(core_axis_name="c", subcore_axis_name="s")

    @functools.partial(
        pl.kernel,
        mesh=mesh,
        out_type=jax.ShapeDtypeStruct((_NW, _LANES), jnp.int32),
        scratch_types=[
            pltpu.VMEM((N,), jnp.float32),
            pltpu.VMEM((_CAP * _LANES + _LANES,), jnp.int32),
            pltpu.VMEM((_LANES,), jnp.int32),
        ],
    )
    def run(x_hbm, out_hbm, rowbuf, candbuf, outbuf):
        wid = lax.axis_index("s") * _info.num_cores + lax.axis_index("c")
        lane = lax.iota(jnp.int32, _LANES)
        rot_idx = [(lane + sh) & (_LANES - 1) for sh in (8, 4, 2, 1)]

        def row_body(rr, acc):
            row = wid * rows_per_w + rr
            pltpu.sync_copy(x_hbm.at[row], rowbuf)
            th, ist = _row_threshold(rowbuf, candbuf, N, rot_idx)
            acc = jnp.where(lane == rr, th, acc)
            return jnp.where(lane == rr + rows_per_w, ist, acc)

        acc = lax.fori_loop(0, rows_per_w, row_body,
                            jnp.zeros((_LANES,), jnp.int32))
        outbuf[...] = acc
        pltpu.sync_copy(outbuf, out_hbm.at[wid])

    return run(x)


def _tc_body(x_ref, t_ref, i_ref, o_ref):
    xv = x_ref[...]
    i = lax.bitcast_convert_type(xv, jnp.int32)
    key = i ^ (lax.shift_right_arithmetic(i, 31) & jnp.int32(0x7FFFFFFF))
    col = lax.broadcasted_iota(jnp.int32, xv.shape, 1)
    keep = jnp.logical_or(
        key > t_ref[...],
        jnp.logical_and(key == t_ref[...], col <= i_ref[...]))
    o_ref[...] = jnp.where(keep, jnp.maximum(xv, 0.0), 0.0)


def kernel(x):
    B, N = x.shape
    rows_per_w = B // _NW
    th2d = _sc_thresholds(x)
    thresh = th2d[:, :rows_per_w].reshape(B, 1)
    icut = th2d[:, rows_per_w:2 * rows_per_w].reshape(B, 1)
    block_b = 8
    return pl.pallas_call(
        _tc_body,
        grid=(B // block_b,),
        in_specs=[pl.BlockSpec((block_b, N), lambda b: (b, 0)),
                  pl.BlockSpec((block_b, 1), lambda b: (b, 0)),
                  pl.BlockSpec((block_b, 1), lambda b: (b, 0))],
        out_specs=pl.BlockSpec((block_b, N), lambda b: (b, 0)),
        out_shape=jax.ShapeDtypeStruct((B, N), x.dtype),
    )(x, thresh, icut)


# speedup vs baseline: 1.2945x; 1.2945x over previous
import functools

import jax
import jax.numpy as jnp
from jax import lax
from jax.experimental import pallas as pl
from jax.experimental.pallas import tpu as pltpu
from jax.experimental.pallas import tpu_sc as plsc

_K = 64
_SIGN = -2147483648
_CAP = 1024
_NSAMP = 512
_RTARG = 6
_LANES = 16

_info = plsc.get_sparse_core_info()
_NW = _info.num_cores * _info.num_subcores


def _key16(ref, off):
    iv = lax.bitcast_convert_type(ref[pl.ds(off, _LANES)], jnp.int32)
    return iv ^ (lax.shift_right_arithmetic(iv, 31) & jnp.int32(0x7FFFFFFF))


def _lane_total(v, rot_idx):
    for idx in rot_idx:
        v = v + v.at[idx].get(mode="promise_in_bounds")
    return v[0]


def _row_threshold(rowbuf, candbuf, n, rot_idx):
    nv = n // _LANES
    one, zero = jnp.int32(1), jnp.int32(0)
    zvec = jnp.zeros((_LANES,), jnp.int32)

    def counts3(load_key, trips, unroll, c3s, c2s, c1s):
        def body(i, st):
            a3, a2, a1 = st
            base = i * (_LANES * unroll)
            for u in range(unroll):
                k = load_key(base + u * _LANES)
                a3 = a3 + jnp.where(k >= c3s, one, zero)
                a2 = a2 + jnp.where(k >= c2s, one, zero)
                a1 = a1 + jnp.where(k >= c1s, one, zero)
            return (a3, a2, a1)
        a3, a2, a1 = lax.fori_loop(0, trips, body, (zvec, zvec, zvec))
        return (_lane_total(a3, rot_idx), _lane_total(a2, rot_idx),
                _lane_total(a1, rot_idx))

    def step2bit(t, st, load_key, ntrips, unroll, stop, kmin):
        upfx, cnt = st
        trips = jnp.where(cnt > stop, ntrips, 0)
        b1 = lax.shift_left(jnp.int32(1), jnp.int32(31) - 2 * t)
        b0 = lax.shift_left(jnp.int32(1), jnp.int32(30) - 2 * t)
        c3 = upfx | b1 | b0
        c2 = upfx | b1
        c1 = upfx | b0
        n3, n2, n1 = counts3(load_key, trips, unroll,
                             c3 ^ _SIGN, c2 ^ _SIGN, c1 ^ _SIGN)
        take3 = n3 >= kmin
        take2 = jnp.logical_and(~take3, n2 >= kmin)
        take1 = jnp.logical_and(~(take3 | take2), n1 >= kmin)
        newp = jnp.where(take3, c3,
                         jnp.where(take2, c2, jnp.where(take1, c1, upfx)))
        newc = jnp.where(take3, n3,
                         jnp.where(take2, n2, jnp.where(take1, n1, cnt)))
        return (newp, newc)

    stride = (nv // (_NSAMP // _LANES)) * _LANES

    def samp_key(off):
        return _key16(rowbuf, (off // _LANES) * stride)

    piv, _ = lax.fori_loop(
        0, 16,
        lambda t, st: step2bit(t, st, samp_key, _NSAMP // _LANES, 1,
                               _RTARG, _RTARG),
        (jnp.int32(0), jnp.int32(_NSAMP)))

    limit = jnp.int32(_CAP * _LANES)
    piv_s = piv ^ _SIGN

    def gbody(g, st):
        off, tot = st
        ks, hs = [], []
        for u in range(4):
            k = _key16(rowbuf, (g * 4 + u) * _LANES)
            m = k >= piv_s
            ks.append(jnp.where(m, k, jnp.int32(_SIGN)))
            hs.append(_lane_total(jnp.where(m, one, zero), rot_idx))
        for u in range(4):
            candbuf[pl.ds(off, _LANES)] = ks[u]
            off = jnp.where(hs[u] > 0,
                            jnp.minimum(off + _LANES, limit), off)
            tot = tot + hs[u]
        return (off, tot)

    off_p, cnt_p = lax.fori_loop(0, nv // 4, gbody,
                                 (jnp.int32(0), jnp.int32(0)))
    ok = jnp.logical_and(cnt_p >= _K, cnt_p <= _CAP)

    def fb_key(off):
        return _key16(rowbuf, off)

    fb_trips = jnp.where(ok, 0, nv // 4)
    upfx_fb, cnt_fb = lax.fori_loop(
        0, 16,
        lambda t, st: step2bit(t, st, fb_key, fb_trips, 4, _CAP, _K),
        (jnp.int32(0), jnp.int32(n)))
    valid_fb = jnp.logical_and(~ok, cnt_fb <= _CAP)
    thr_fb = upfx_fb ^ _SIGN

    def cbody(i, off):
        k = _key16(rowbuf, i * _LANES)
        m = k >= thr_fb
        candbuf[pl.ds(off, _LANES)] = jnp.where(m, k, jnp.int32(_SIGN))
        hits = _lane_total(jnp.where(m, one, zero), rot_idx)
        return jnp.where(hits > 0, off + _LANES, off)

    off_fb = lax.fori_loop(0, jnp.where(valid_fb, nv, 0), cbody,
                           jnp.int32(0))

    use_fb = ~ok
    upfx_in = jnp.where(use_fb, upfx_fb, jnp.int32(0))
    cnt_in = jnp.where(use_fb, cnt_fb, cnt_p)
    off_sel = jnp.where(use_fb, off_fb, off_p)
    sv = off_sel // _LANES
    sv = jnp.where(jnp.logical_and(use_fb, ~valid_fb), 0, sv)
    candbuf[pl.ds(off_sel, _LANES)] = jnp.full((_LANES,), _SIGN, jnp.int32)

    def cand_key(off):
        return candbuf[pl.ds(off, _LANES)]

    upfx, fcnt = lax.fori_loop(
        0, 16,
        lambda t, st: step2bit(t, st, cand_key, (sv + 1) // 2, 2, _K, _K),
        (upfx_in, cnt_in))

    pivhit = jnp.logical_and(ok, cnt_p == _K)
    upfx = jnp.where(pivhit, piv, upfx)
    fcnt = jnp.where(pivhit, cnt_p, fcnt)
    thr_s = upfx ^ _SIGN

    tie = fcnt > _K
    tie_trips = jnp.where(tie, nv // 4, 0)

    def gtbody(i, a):
        base = i * (_LANES * 4)
        for u in range(4):
            k = _key16(rowbuf, base + u * _LANES)
            a = a + jnp.where(k > thr_s, one, zero)
        return a

    ngt = _lane_total(lax.fori_loop(0, tie_trips, gtbody, zvec), rot_idx)
    req = _K - ngt

    lane = lax.iota(jnp.int32, _LANES)

    def idx_step(t, vpfx):
        b1 = lax.shift_left(jnp.int32(1), jnp.int32(15) - 2 * t)
        b0 = lax.shift_left(jnp.int32(1), jnp.int32(14) - 2 * t)
        c3, c2, c1 = vpfx | b1 | b0, vpfx | b1, vpfx | b0

        def body(i, st):
            a3, a2, a1 = st
            base = i * (_LANES * 4)
            for u in range(4):
                off = base + u * _LANES
                k = _key16(rowbuf, off)
                eq = k == thr_s
                rv = jnp.int32(n - 1) - (off + lane)
                a3 = a3 + jnp.where(jnp.logical_and(eq, rv >= c3), one, zero)
                a2 = a2 + jnp.where(jnp.logical_and(eq, rv >= c2), one, zero)
                a1 = a1 + jnp.where(jnp.logical_and(eq, rv >= c1), one, zero)
            return (a3, a2, a1)

        a3, a2, a1 = lax.fori_loop(0, tie_trips, body, (zvec, zvec, zvec))
        n3 = _lane_total(a3, rot_idx)
        n2 = _lane_total(a2, rot_idx)
        n1 = _lane_total(a1, rot_idx)
        take3 = n3 >= req
        take2 = jnp.logical_and(~take3, n2 >= req)
        take1 = jnp.logical_and(~(take3 | take2), n1 >= req)
        return jnp.where(take3, c3,
                         jnp.where(take2, c2, jnp.where(take1, c1, vpfx)))

    vstar = lax.fori_loop(0, 8, idx_step, jnp.int32(0))
    istar = jnp.where(tie, jnp.int32(n - 1) - vstar, jnp.int32(n - 1))

    return thr_s, istar


def _sc_thresholds(x):
    B, N = x.shape
    rows_per_w = B // _NW
    mesh = plsc.VectorSubcoreMesh(core_axis_name="c", subcore_axis_name="s")

    @functools.partial(
        pl.kernel,
        mesh=mesh,
        out_type=jax.ShapeDtypeStruct((_NW, _LANES), jnp.int32),
        scratch_types=[
            pltpu.VMEM((N,), jnp.float32),
            pltpu.VMEM((_CAP * _LANES + _LANES,), jnp.int32),
            pltpu.VMEM((_LANES,), jnp.int32),
        ],
    )
    def run(x_hbm, out_hbm, rowbuf, candbuf, outbuf):
        wid = lax.axis_index("s") * _info.num_cores + lax.axis_index("c")
        lane = lax.iota(jnp.int32, _LANES)
        rot_idx = [(lane + sh) & (_LANES - 1) for sh in (8, 4, 2, 1)]

        def row_body(rr, acc):
            row = wid * rows_per_w + rr
            pltpu.sync_copy(x_hbm.at[row], rowbuf)
            th, ist = _row_threshold(rowbuf, candbuf, N, rot_idx)
            acc = jnp.where(lane == rr, th, acc)
            return jnp.where(lane == rr + rows_per_w, ist, acc)

        acc = lax.fori_loop(0, rows_per_w, row_body,
                            jnp.zeros((_LANES,), jnp.int32))
        outbuf[...] = acc
        pltpu.sync_copy(outbuf, out_hbm.at[wid])

    return run(x)


def _tc_body(x_ref, t_ref, i_ref, o_ref):
    xv = x_ref[...]
    i = lax.bitcast_convert_type(xv, jnp.int32)
    key = i ^ (lax.shift_right_arithmetic(i, 31) & jnp.int32(0x7FFFFFFF))
    col = lax.broadcasted_iota(jnp.int32, xv.shape, 1)
    keep = jnp.logical_or(
        key > t_ref[...],
        jnp.logical_and(key == t_ref[...], col <= i_ref[...]))
    o_ref[...] = jnp.where(keep, jnp.maximum(xv, 0.0), 0.0)


def kernel(x):
    B, N = x.shape
    rows_per_w = B // _NW
    th2d = _sc_thresholds(x)
    thresh = th2d[:, :rows_per_w].reshape(B, 1)
    icut = th2d[:, rows_per_w:2 * rows_per_w].reshape(B, 1)
    block_b = 8
    return pl.pallas_call(
        _tc_body,
        grid=(B // block_b,),
        in_specs=[pl.BlockSpec((block_b, N), lambda b: (b, 0)),
                  pl.BlockSpec((block_b, 1), lambda b: (b, 0)),
                  pl.BlockSpec((block_b, 1), lambda b: (b, 0))],
        out_specs=pl.BlockSpec((block_b, N), lambda b: (b, 0)),
        out_shape=jax.ShapeDtypeStruct((B, N), x.dtype),
    )(x, thresh, icut)


# speedup vs baseline: 1.3381x; 1.0337x over previous
import functools

import jax
import jax.numpy as jnp
from jax import lax
from jax.experimental import pallas as pl
from jax.experimental.pallas import tpu as pltpu
from jax.experimental.pallas import tpu_sc as plsc

_K = 64
_SIGN = -2147483648
_CAP = 1024
_NSAMP = 512
_RTARG = 6
_LANES = 16

_info = plsc.get_sparse_core_info()
_NW = _info.num_cores * _info.num_subcores


def _key16(ref, off):
    iv = lax.bitcast_convert_type(ref[pl.ds(off, _LANES)], jnp.int32)
    return iv ^ (lax.shift_right_arithmetic(iv, 31) & jnp.int32(0x7FFFFFFF))


def _lane_total(v, rot_idx):
    for idx in rot_idx:
        v = v + v.at[idx].get(mode="promise_in_bounds")
    return v[0]


def _row_threshold(rowbuf, candbuf, n, rot_idx):
    nv = n // _LANES
    one, zero = jnp.int32(1), jnp.int32(0)
    zvec = jnp.zeros((_LANES,), jnp.int32)

    def counts3(load_key, trips, unroll, c3s, c2s, c1s):
        def body(i, st):
            a3, a2, a1 = st
            base = i * (_LANES * unroll)
            for u in range(unroll):
                k = load_key(base + u * _LANES)
                a3 = a3 + jnp.where(k >= c3s, one, zero)
                a2 = a2 + jnp.where(k >= c2s, one, zero)
                a1 = a1 + jnp.where(k >= c1s, one, zero)
            return (a3, a2, a1)
        a3, a2, a1 = lax.fori_loop(0, trips, body, (zvec, zvec, zvec))
        return (_lane_total(a3, rot_idx), _lane_total(a2, rot_idx),
                _lane_total(a1, rot_idx))

    def step2bit(t, st, load_key, ntrips, unroll, stop, kmin):
        upfx, cnt = st
        trips = jnp.where(cnt > stop, ntrips, 0)
        b1 = lax.shift_left(jnp.int32(1), jnp.int32(31) - 2 * t)
        b0 = lax.shift_left(jnp.int32(1), jnp.int32(30) - 2 * t)
        c3 = upfx | b1 | b0
        c2 = upfx | b1
        c1 = upfx | b0
        n3, n2, n1 = counts3(load_key, trips, unroll,
                             c3 ^ _SIGN, c2 ^ _SIGN, c1 ^ _SIGN)
        take3 = n3 >= kmin
        take2 = jnp.logical_and(~take3, n2 >= kmin)
        take1 = jnp.logical_and(~(take3 | take2), n1 >= kmin)
        newp = jnp.where(take3, c3,
                         jnp.where(take2, c2, jnp.where(take1, c1, upfx)))
        newc = jnp.where(take3, n3,
                         jnp.where(take2, n2, jnp.where(take1, n1, cnt)))
        return (newp, newc)

    stride = (nv // (_NSAMP // _LANES)) * _LANES

    def samp_key(off):
        return _key16(rowbuf, (off // _LANES) * stride)

    piv, _ = lax.fori_loop(
        0, 16,
        lambda t, st: step2bit(t, st, samp_key, _NSAMP // _LANES, 1,
                               _RTARG, _RTARG),
        (jnp.int32(0), jnp.int32(_NSAMP)))

    limit = jnp.int32(_CAP * _LANES)
    piv_s = piv ^ _SIGN

    def gbody(g, st):
        off, tot = st
        ks, hs = [], []
        for u in range(4):
            k = _key16(rowbuf, (g * 4 + u) * _LANES)
            m = k >= piv_s
            ks.append(jnp.where(m, k, jnp.int32(_SIGN)))
            hs.append(_lane_total(jnp.where(m, one, zero), rot_idx))
        for u in range(4):
            candbuf[pl.ds(off, _LANES)] = ks[u]
            off = jnp.where(hs[u] > 0,
                            jnp.minimum(off + _LANES, limit), off)
            tot = tot + hs[u]
        return (off, tot)

    off_p, cnt_p = lax.fori_loop(0, nv // 4, gbody,
                                 (jnp.int32(0), jnp.int32(0)))
    ok = jnp.logical_and(cnt_p >= _K, cnt_p <= _CAP)

    def fb_key(off):
        return _key16(rowbuf, off)

    fb_trips = jnp.where(ok, 0, nv // 4)
    upfx_fb, cnt_fb = lax.fori_loop(
        0, 16,
        lambda t, st: step2bit(t, st, fb_key, fb_trips, 4, _CAP, _K),
        (jnp.int32(0), jnp.int32(n)))
    valid_fb = jnp.logical_and(~ok, cnt_fb <= _CAP)
    thr_fb = upfx_fb ^ _SIGN

    def cbody(i, off):
        k = _key16(rowbuf, i * _LANES)
        m = k >= thr_fb
        candbuf[pl.ds(off, _LANES)] = jnp.where(m, k, jnp.int32(_SIGN))
        hits = _lane_total(jnp.where(m, one, zero), rot_idx)
        return jnp.where(hits > 0, off + _LANES, off)

    off_fb = lax.fori_loop(0, jnp.where(valid_fb, nv, 0), cbody,
                           jnp.int32(0))

    use_fb = ~ok
    upfx_in = jnp.where(use_fb, upfx_fb, jnp.int32(0))
    cnt_in = jnp.where(use_fb, cnt_fb, cnt_p)
    off_sel = jnp.where(use_fb, off_fb, off_p)
    sv = off_sel // _LANES
    sv = jnp.where(jnp.logical_and(use_fb, ~valid_fb), 0, sv)
    for p in range(3):
        candbuf[pl.ds(off_sel + p * _LANES, _LANES)] = jnp.full(
            (_LANES,), _SIGN, jnp.int32)

    def cand_key(off):
        return candbuf[pl.ds(off, _LANES)]

    upfx, fcnt = lax.fori_loop(
        0, 16,
        lambda t, st: step2bit(t, st, cand_key, (sv + 3) // 4, 4, _K, _K),
        (upfx_in, cnt_in))

    pivhit = jnp.logical_and(ok, cnt_p == _K)
    upfx = jnp.where(pivhit, piv, upfx)
    fcnt = jnp.where(pivhit, cnt_p, fcnt)
    thr_s = upfx ^ _SIGN

    tie = fcnt > _K
    tie_trips = jnp.where(tie, nv // 4, 0)

    def gtbody(i, a):
        base = i * (_LANES * 4)
        for u in range(4):
            k = _key16(rowbuf, base + u * _LANES)
            a = a + jnp.where(k > thr_s, one, zero)
        return a

    ngt = _lane_total(lax.fori_loop(0, tie_trips, gtbody, zvec), rot_idx)
    req = _K - ngt

    lane = lax.iota(jnp.int32, _LANES)

    def idx_step(t, vpfx):
        b1 = lax.shift_left(jnp.int32(1), jnp.int32(15) - 2 * t)
        b0 = lax.shift_left(jnp.int32(1), jnp.int32(14) - 2 * t)
        c3, c2, c1 = vpfx | b1 | b0, vpfx | b1, vpfx | b0

        def body(i, st):
            a3, a2, a1 = st
            base = i * (_LANES * 4)
            for u in range(4):
                off = base + u * _LANES
                k = _key16(rowbuf, off)
                eq = k == thr_s
                rv = jnp.int32(n - 1) - (off + lane)
                a3 = a3 + jnp.where(jnp.logical_and(eq, rv >= c3), one, zero)
                a2 = a2 + jnp.where(jnp.logical_and(eq, rv >= c2), one, zero)
                a1 = a1 + jnp.where(jnp.logical_and(eq, rv >= c1), one, zero)
            return (a3, a2, a1)

        a3, a2, a1 = lax.fori_loop(0, tie_trips, body, (zvec, zvec, zvec))
        n3 = _lane_total(a3, rot_idx)
        n2 = _lane_total(a2, rot_idx)
        n1 = _lane_total(a1, rot_idx)
        take3 = n3 >= req
        take2 = jnp.logical_and(~take3, n2 >= req)
        take1 = jnp.logical_and(~(take3 | take2), n1 >= req)
        return jnp.where(take3, c3,
                         jnp.where(take2, c2, jnp.where(take1, c1, vpfx)))

    vstar = lax.fori_loop(0, 8, idx_step, jnp.int32(0))
    istar = jnp.where(tie, jnp.int32(n - 1) - vstar, jnp.int32(n - 1))

    return thr_s, istar


def _sc_thresholds(x):
    B, N = x.shape
    rows_per_w = B // _NW
    mesh = plsc.VectorSubcoreMesh(core_axis_name="c", subcore_axis_name="s")

    @functools.partial(
        pl.kernel,
        mesh=mesh,
        out_type=jax.ShapeDtypeStruct((_NW, _LANES), jnp.int32),
        scratch_types=[
            pltpu.VMEM((N,), jnp.float32),
            pltpu.VMEM((_CAP * _LANES + 4 * _LANES,), jnp.int32),
            pltpu.VMEM((_LANES,), jnp.int32),
        ],
    )
    def run(x_hbm, out_hbm, rowbuf, candbuf, outbuf):
        wid = lax.axis_index("s") * _info.num_cores + lax.axis_index("c")
        lane = lax.iota(jnp.int32, _LANES)
        rot_idx = [(lane + sh) & (_LANES - 1) for sh in (8, 4, 2, 1)]

        def row_body(rr, acc):
            row = wid * rows_per_w + rr
            pltpu.sync_copy(x_hbm.at[row], rowbuf)
            th, ist = _row_threshold(rowbuf, candbuf, N, rot_idx)
            acc = jnp.where(lane == rr, th, acc)
            return jnp.where(lane == rr + rows_per_w, ist, acc)

        acc = lax.fori_loop(0, rows_per_w, row_body,
                            jnp.zeros((_LANES,), jnp.int32))
        outbuf[...] = acc
        pltpu.sync_copy(outbuf, out_hbm.at[wid])

    return run(x)


def _tc_body(x_ref, t_ref, i_ref, o_ref):
    xv = x_ref[...]
    i = lax.bitcast_convert_type(xv, jnp.int32)
    key = i ^ (lax.shift_right_arithmetic(i, 31) & jnp.int32(0x7FFFFFFF))
    col = lax.broadcasted_iota(jnp.int32, xv.shape, 1)
    keep = jnp.logical_or(
        key > t_ref[...],
        jnp.logical_and(key == t_ref[...], col <= i_ref[...]))
    o_ref[...] = jnp.where(keep, jnp.maximum(xv, 0.0), 0.0)


def kernel(x):
    B, N = x.shape
    rows_per_w = B // _NW
    th2d = _sc_thresholds(x)
    thresh = th2d[:, :rows_per_w].reshape(B, 1)
    icut = th2d[:, rows_per_w:2 * rows_per_w].reshape(B, 1)
    block_b = 8
    return pl.pallas_call(
        _tc_body,
        grid=(B // block_b,),
        in_specs=[pl.BlockSpec((block_b, N), lambda b: (b, 0)),
                  pl.BlockSpec((block_b, 1), lambda b: (b, 0)),
                  pl.BlockSpec((block_b, 1), lambda b: (b, 0))],
        out_specs=pl.BlockSpec((block_b, N), lambda b: (b, 0)),
        out_shape=jax.ShapeDtypeStruct((B, N), x.dtype),
    )(x, thresh, icut)
